# Initial kernel scaffold; baseline (speedup 1.0000x reference)
import jax
import jax.numpy as jnp
from jax.experimental import pallas as pl


def kernel(x, enc1_w, enc1_b, enc2_w, enc2_b, enc3_w, enc3_b, dec1_w, dec1_b, dec2_w, dec2_b, dec3_w, dec3_b):
    raise NotImplementedError("write your pallas kernel here")


# speedup vs baseline: 1.7683x; 1.7683x over previous
import functools

import jax
import jax.numpy as jnp
from jax.experimental import pallas as pl
from jax.experimental.pallas import tpu as pltpu


def _enc_w(w, b):
    cout, cin = w.shape[0], w.shape[1]
    wm = jnp.transpose(w, (2, 3, 1, 0)).reshape(9 * cin, cout)
    return wm.astype(jnp.float32), b.astype(jnp.float32).reshape(1, cout)


def _dec_w(w, b):
    cin, cout = w.shape[0], w.shape[1]
    z = jnp.zeros((cin, cout), w.dtype)
    rows = []
    for ty in range(2):
        for tx in range(2):
            blocks = []
            for py in range(2):
                for px in range(2):
                    kh, kw = py - 2 * ty + 1, px - 2 * tx + 1
                    ok = 0 <= kh <= 2 and 0 <= kw <= 2
                    blocks.append(w[:, :, kh, kw] if ok else z)
            rows.append(jnp.concatenate(blocks, axis=1))
    wm = jnp.concatenate(rows, axis=0).astype(jnp.float32)
    bb = jnp.tile(b.astype(jnp.float32), 4).reshape(1, 4 * cout)
    return wm, bb


def _enc_body(prev_ref, cur_ref, next_ref, w_ref, b_ref, o_ref, *, wd, nb):
    blk = pl.program_id(1)
    bsz, cin = cur_ref.shape[1], cur_ref.shape[2]
    cout = o_ref.shape[2]

    cur = cur_ref[0]
    prev_t = jnp.where(blk == 0, 0.0, prev_ref[0, bsz - wd:, :])
    next_h = jnp.where(blk == nb - 1, 0.0, next_ref[0, :wd, :])
    zpad = jnp.zeros((8, cin), jnp.float32)
    xfull = jnp.concatenate([zpad, prev_t, cur, next_h, zpad], axis=0)
    pos = wd + 8

    wcol = jax.lax.rem(jax.lax.broadcasted_iota(jnp.int32, (bsz, 1), 0), wd)
    m_l, m_r = wcol == 0, wcol == wd - 1

    pieces = []
    for dy in (-1, 0, 1):
        for dx in (-1, 0, 1):
            s = pos + dy * wd + dx
            xs = xfull[s:s + bsz, :]
            if dx == -1:
                xs = jnp.where(m_l, 0.0, xs)
            elif dx == 1:
                xs = jnp.where(m_r, 0.0, xs)
            pieces.append(xs)
    pat = jnp.concatenate(pieces, axis=1)

    y = jnp.dot(pat, w_ref[...], preferred_element_type=jnp.float32)
    y = jnp.maximum(y + b_ref[...], 0.0)

    rbh = bsz // wd
    m = jnp.max(y.reshape(bsz // 2, 2, cout), axis=1)
    m = jnp.max(m.reshape(rbh // 2, 2, (wd // 2), cout), axis=1)
    o_ref[0] = m.reshape(bsz // 4, cout)


def _dec_body(cur_ref, next_ref, w_ref, b_ref, o_ref, *, wd, nb, act):
    blk = pl.program_id(1)
    bsz, cin = cur_ref.shape[1], cur_ref.shape[2]

    next_h = jnp.where(blk == nb - 1, 0.0, next_ref[0, :wd, :])
    zpad = jnp.zeros((8, cin), jnp.float32)
    xfull = jnp.concatenate([cur_ref[0], next_h, zpad], axis=0)

    wcol = jax.lax.rem(jax.lax.broadcasted_iota(jnp.int32, (bsz, 1), 0), wd)
    m_r = wcol == wd - 1

    pieces = []
    for ty in (0, 1):
        for tx in (0, 1):
            xs = xfull[ty * wd + tx: ty * wd + tx + bsz, :]
            if tx == 1:
                xs = jnp.where(m_r, 0.0, xs)
            pieces.append(xs)
    pat = jnp.concatenate(pieces, axis=1)

    r = jnp.dot(pat, w_ref[...], preferred_element_type=jnp.float32)
    r = r + b_ref[...]
    if act == "relu":
        r = jnp.maximum(r, 0.0)
    else:
        r = pl.reciprocal(1.0 + jnp.exp(-r), approx=True)
    o_ref[0] = r


def _pick_nb(hw, cin, cout):
    budget = 6 * 1024 * 1024 // 4
    per_row = max(cin, 128) + 9 * max(cin, 16) + max(cout, 128)
    nb = 1
    while hw // nb * per_row > budget and nb < 16:
        nb *= 2
    return nb


def _enc_layer(x, w, b, wd):
    n, hw, cin = x.shape
    cout = w.shape[0]
    wm, bb = _enc_w(w, b)
    nb = _pick_nb(hw, cin, cout)
    bsz = hw // nb
    body = functools.partial(_enc_body, wd=wd, nb=nb)
    return pl.pallas_call(
        body,
        out_shape=jax.ShapeDtypeStruct((n, hw // 4, cout), jnp.float32),
        grid=(n, nb),
        in_specs=[
            pl.BlockSpec((1, bsz, cin), lambda i, j: (i, jnp.maximum(j - 1, 0), 0)),
            pl.BlockSpec((1, bsz, cin), lambda i, j: (i, j, 0)),
            pl.BlockSpec((1, bsz, cin), lambda i, j: (i, jnp.minimum(j + 1, nb - 1), 0)),
            pl.BlockSpec((9 * cin, cout), lambda i, j: (0, 0)),
            pl.BlockSpec((1, cout), lambda i, j: (0, 0)),
        ],
        out_specs=pl.BlockSpec((1, bsz // 4, cout), lambda i, j: (i, j, 0)),
        compiler_params=pltpu.CompilerParams(
            dimension_semantics=("parallel", "arbitrary")),
    )(x, x, x, wm, bb)


def _dec_layer(x, w, b, wd, act):
    n, hw, cin = x.shape
    cout = w.shape[1]
    wm, bb = _dec_w(w, b)
    nb = _pick_nb(hw, cin, 4 * cout)
    bsz = hw // nb
    body = functools.partial(_dec_body, wd=wd, nb=nb, act=act)
    out_spec = pl.BlockSpec((1, bsz, 4 * cout), lambda i, j: (i, j, 0))
    return pl.pallas_call(
        body,
        out_shape=jax.ShapeDtypeStruct((n, hw, 4 * cout), jnp.float32),
        grid=(n, nb),
        in_specs=[
            pl.BlockSpec((1, bsz, cin), lambda i, j: (i, j, 0)),
            pl.BlockSpec((1, bsz, cin), lambda i, j: (i, jnp.minimum(j + 1, nb - 1), 0)),
            pl.BlockSpec((4 * cin, 4 * cout), lambda i, j: (0, 0)),
            pl.BlockSpec((1, 4 * cout), lambda i, j: (0, 0)),
        ],
        out_specs=out_spec,
        compiler_params=pltpu.CompilerParams(
            dimension_semantics=("parallel", "arbitrary")),
    )(x, x, wm, bb)


def _interleave(r, hgrid, wgrid):
    n = r.shape[0]
    co = r.shape[2] // 4
    y = r.reshape(n, hgrid, wgrid, 2, 2, co)
    y = jnp.transpose(y, (0, 1, 3, 2, 4, 5))
    return y.reshape(n, 4 * hgrid * wgrid, co)


def kernel(x, enc1_w, enc1_b, enc2_w, enc2_b, enc3_w, enc3_b,
           dec1_w, dec1_b, dec2_w, dec2_b, dec3_w, dec3_b):
    n, _, h, w = x.shape
    xr = jnp.transpose(x, (0, 2, 3, 1)).reshape(n, h * w, x.shape[1])

    e1 = _enc_layer(xr, enc1_w, enc1_b, wd=w)
    e2 = _enc_layer(e1, enc2_w, enc2_b, wd=w // 2)
    e3 = _enc_layer(e2, enc3_w, enc3_b, wd=w // 4)

    d1 = _dec_layer(e3, dec1_w, dec1_b, wd=w // 8, act="relu")
    d1 = _interleave(d1, h // 8, w // 8)
    d2 = _dec_layer(d1, dec2_w, dec2_b, wd=w // 4, act="relu")
    d2 = _interleave(d2, h // 4, w // 4)
    d3 = _dec_layer(d2, dec3_w, dec3_b, wd=w // 2, act="sigmoid")

    h2, w2, co = h // 2, w // 2, dec3_w.shape[1]
    y = d3.reshape(n, h2, w2, 2, 2, co)
    return jnp.transpose(y, (0, 5, 1, 3, 2, 4)).reshape(n, co, h, w)


# speedup vs baseline: 4.2785x; 2.4196x over previous
import functools

import jax
import jax.numpy as jnp
from jax.experimental import pallas as pl
from jax.experimental.pallas import tpu as pltpu


def _rup(x, m):
    return (x + m - 1) // m * m


def _enc_w(w, b, cpad):
    cout, cin = w.shape[0], w.shape[1]
    wm = jnp.transpose(w, (2, 3, 1, 0)).astype(jnp.float32)
    wm = jnp.pad(wm, ((0, 0), (0, 0), (0, cpad - cin), (0, 0)))
    return wm.reshape(9 * cpad, cout), b.astype(jnp.float32).reshape(1, cout)


def _dec_w(w, b):
    cin, cout = w.shape[0], w.shape[1]
    z = jnp.zeros((cin, cout), w.dtype)
    rows = []
    for ty in range(2):
        for tx in range(2):
            blocks = []
            for py in range(2):
                for px in range(2):
                    kh, kw = py - 2 * ty + 1, px - 2 * tx + 1
                    ok = 0 <= kh <= 2 and 0 <= kw <= 2
                    blocks.append(w[:, :, kh, kw] if ok else z)
            rows.append(jnp.concatenate(blocks, axis=1))
    wm = jnp.concatenate(rows, axis=0).astype(jnp.float32)
    bb = jnp.tile(b.astype(jnp.float32), 4).reshape(1, 4 * cout)
    return wm, bb


def _enc_plane_body(prev_ref, cur_ref, next_ref, w_ref, b_ref, o_ref, *,
                    wd, nb, halo, cpad):
    j = pl.program_id(1)
    c, x = cur_ref.shape[1], cur_ref.shape[2]
    cout = o_ref.shape[1]

    prev_t = jnp.where(j == 0, 0.0, prev_ref[0, :, x - halo:])
    next_h = jnp.where(j == nb - 1, 0.0, next_ref[0, :, :halo])
    base = jnp.concatenate([prev_t, cur_ref[0], next_h], axis=1)
    if c < cpad:
        base = jnp.concatenate(
            [base, jnp.zeros((cpad - c, x + 2 * halo), jnp.float32)], axis=0)

    wcol = jax.lax.rem(jax.lax.broadcasted_iota(jnp.int32, (1, x), 1), wd)
    m_l, m_r = wcol == 0, wcol == wd - 1

    pieces = []
    for dy in (-1, 0, 1):
        for dx in (-1, 0, 1):
            s = halo + dy * wd + dx
            xs = base[:, s:s + x]
            if dx == -1:
                xs = jnp.where(m_l, 0.0, xs)
            elif dx == 1:
                xs = jnp.where(m_r, 0.0, xs)
            pieces.append(xs)
    pat = jnp.concatenate(pieces, axis=0)

    y = jax.lax.dot_general(pat, w_ref[...], (((0,), (0,)), ((), ())),
                            preferred_element_type=jnp.float32)
    y = jnp.maximum(y + b_ref[...], 0.0)

    m = jnp.max(y.reshape(x // 2, 2, cout), axis=1)
    m = jnp.max(m.reshape(x // (2 * wd), 2, wd // 2, cout), axis=1)
    o_ref[0] = jnp.transpose(m.reshape(x // 4, cout), (1, 0))


def _dec_plane_body(cur_ref, w_ref, b_ref, o_ref, *, wd):
    c, x = cur_ref.shape[1], cur_ref.shape[2]
    base = jnp.concatenate([cur_ref[0], jnp.zeros((c, 128), jnp.float32)],
                           axis=1)
    wcol = jax.lax.rem(jax.lax.broadcasted_iota(jnp.int32, (1, x), 1), wd)
    m_r = wcol == wd - 1

    pieces = []
    for ty in (0, 1):
        for tx in (0, 1):
            xs = base[:, ty * wd + tx: ty * wd + tx + x]
            if tx == 1:
                xs = jnp.where(m_r, 0.0, xs)
            pieces.append(xs)
    pat = jnp.concatenate(pieces, axis=0)

    r = jax.lax.dot_general(pat, w_ref[...], (((0,), (0,)), ((), ())),
                            preferred_element_type=jnp.float32)
    o_ref[0] = jnp.maximum(r + b_ref[...], 0.0)


def _dec_rows_body(cur_ref, next_ref, w_ref, b_ref, o_ref, *, wd, nb, act):
    blk = pl.program_id(1)
    bsz, cin = cur_ref.shape[1], cur_ref.shape[2]

    next_h = jnp.where(blk == nb - 1, 0.0, next_ref[0, :wd, :])
    zpad = jnp.zeros((8, cin), jnp.float32)
    xfull = jnp.concatenate([cur_ref[0], next_h, zpad], axis=0)

    wcol = jax.lax.rem(jax.lax.broadcasted_iota(jnp.int32, (bsz, 1), 0), wd)
    m_r = wcol == wd - 1

    pieces = []
    for ty in (0, 1):
        for tx in (0, 1):
            xs = xfull[ty * wd + tx: ty * wd + tx + bsz, :]
            if tx == 1:
                xs = jnp.where(m_r, 0.0, xs)
            pieces.append(xs)
    pat = jnp.concatenate(pieces, axis=1)

    r = jnp.dot(pat, w_ref[...], preferred_element_type=jnp.float32)
    r = r + b_ref[...]
    if act == "relu":
        r = jnp.maximum(r, 0.0)
    else:
        r = pl.reciprocal(1.0 + jnp.exp(-r), approx=True)
    o_ref[0] = r


def _enc_layer(x, w, b, wd, nb):
    n, c, hw = x.shape
    cout = w.shape[0]
    cpad = _rup(c, 8)
    halo = _rup(wd + 1, 128)
    wm, bb = _enc_w(w, b, cpad)
    xsz = hw // nb
    body = functools.partial(_enc_plane_body, wd=wd, nb=nb, halo=halo,
                             cpad=cpad)
    return pl.pallas_call(
        body,
        out_shape=jax.ShapeDtypeStruct((n, cout, hw // 4), jnp.float32),
        grid=(n, nb),
        in_specs=[
            pl.BlockSpec((1, c, xsz), lambda i, j: (i, 0, jnp.maximum(j - 1, 0))),
            pl.BlockSpec((1, c, xsz), lambda i, j: (i, 0, j)),
            pl.BlockSpec((1, c, xsz), lambda i, j: (i, 0, jnp.minimum(j + 1, nb - 1))),
            pl.BlockSpec((9 * cpad, cout), lambda i, j: (0, 0)),
            pl.BlockSpec((1, cout), lambda i, j: (0, 0)),
        ],
        out_specs=pl.BlockSpec((1, cout, xsz // 4), lambda i, j: (i, 0, j)),
        compiler_params=pltpu.CompilerParams(
            dimension_semantics=("parallel", "arbitrary")),
    )(x, x, x, wm, bb)


def _dec_plane_layer(x, w, b, wd):
    n, c, hw = x.shape
    cout = w.shape[1]
    wm, bb = _dec_w(w, b)
    body = functools.partial(_dec_plane_body, wd=wd)
    return pl.pallas_call(
        body,
        out_shape=jax.ShapeDtypeStruct((n, hw, 4 * cout), jnp.float32),
        grid=(n,),
        in_specs=[
            pl.BlockSpec((1, c, hw), lambda i: (i, 0, 0)),
            pl.BlockSpec((4 * c, 4 * cout), lambda i: (0, 0)),
            pl.BlockSpec((1, 4 * cout), lambda i: (0, 0)),
        ],
        out_specs=pl.BlockSpec((1, hw, 4 * cout), lambda i: (i, 0, 0)),
        compiler_params=pltpu.CompilerParams(
            dimension_semantics=("parallel",)),
    )(x, wm, bb)


def _dec_rows_layer(x, w, b, wd, act, nb):
    n, hw, cin = x.shape
    cout = w.shape[1]
    wm, bb = _dec_w(w, b)
    bsz = hw // nb
    body = functools.partial(_dec_rows_body, wd=wd, nb=nb, act=act)
    return pl.pallas_call(
        body,
        out_shape=jax.ShapeDtypeStruct((n, hw, 4 * cout), jnp.float32),
        grid=(n, nb),
        in_specs=[
            pl.BlockSpec((1, bsz, cin), lambda i, j: (i, j, 0)),
            pl.BlockSpec((1, bsz, cin), lambda i, j: (i, jnp.minimum(j + 1, nb - 1), 0)),
            pl.BlockSpec((4 * cin, 4 * cout), lambda i, j: (0, 0)),
            pl.BlockSpec((1, 4 * cout), lambda i, j: (0, 0)),
        ],
        out_specs=pl.BlockSpec((1, bsz, 4 * cout), lambda i, j: (i, j, 0)),
        compiler_params=pltpu.CompilerParams(
            dimension_semantics=("parallel", "arbitrary")),
    )(x, x, wm, bb)


def _interleave(r, hgrid, wgrid):
    n = r.shape[0]
    co = r.shape[2] // 4
    y = r.reshape(n, hgrid, wgrid, 2, 2, co)
    y = jnp.transpose(y, (0, 1, 3, 2, 4, 5))
    return y.reshape(n, 4 * hgrid * wgrid, co)


def kernel(x, enc1_w, enc1_b, enc2_w, enc2_b, enc3_w, enc3_b,
           dec1_w, dec1_b, dec2_w, dec2_b, dec3_w, dec3_b):
    n, cin, h, w = x.shape
    xp = x.reshape(n, cin, h * w)

    e1 = _enc_layer(xp, enc1_w, enc1_b, wd=w, nb=4)
    e2 = _enc_layer(e1, enc2_w, enc2_b, wd=w // 2, nb=2)
    e3 = _enc_layer(e2, enc3_w, enc3_b, wd=w // 4, nb=1)

    d1 = _dec_plane_layer(e3, dec1_w, dec1_b, wd=w // 8)
    d1 = _interleave(d1, h // 8, w // 8)
    d2 = _dec_rows_layer(d1, dec2_w, dec2_b, wd=w // 4, act="relu", nb=1)
    d2 = _interleave(d2, h // 4, w // 4)
    d3 = _dec_rows_layer(d2, dec3_w, dec3_b, wd=w // 2, act="sigmoid", nb=4)

    h2, w2, co = h // 2, w // 2, dec3_w.shape[1]
    y = d3.reshape(n, h2, w2, 2, 2, co)
    return jnp.transpose(y, (0, 5, 1, 3, 2, 4)).reshape(n, co, h, w)


# speedup vs baseline: 6.6171x; 1.5466x over previous
import functools

import jax
import jax.numpy as jnp
from jax.experimental import pallas as pl
from jax.experimental.pallas import tpu as pltpu


def _rup(x, m):
    return (x + m - 1) // m * m


def _enc_w(w, b, cpad):
    cout, cin = w.shape[0], w.shape[1]
    wm = jnp.transpose(w, (2, 3, 1, 0)).astype(jnp.float32)
    wm = jnp.pad(wm, ((0, 0), (0, 0), (0, cpad - cin), (0, 0)))
    return wm.reshape(9 * cpad, cout), b.astype(jnp.float32).reshape(cout, 1)


def _dec_w(w, b):
    cin, cout = w.shape[0], w.shape[1]
    z = jnp.zeros((cin, cout), w.dtype)
    rows = []
    for ty in range(2):
        for tx in range(2):
            blocks = []
            for py in range(2):
                for px in range(2):
                    kh, kw = py - 2 * ty + 1, px - 2 * tx + 1
                    ok = 0 <= kh <= 2 and 0 <= kw <= 2
                    blocks.append(w[:, :, kh, kw] if ok else z)
            rows.append(jnp.concatenate(blocks, axis=1))
    wm = jnp.concatenate(rows, axis=0).astype(jnp.float32)
    bb = jnp.tile(b.astype(jnp.float32), 4).reshape(1, 4 * cout)
    return wm, bb


def _enc_plane_body(prev_ref, cur_ref, next_ref, w_ref, b_ref, o_ref, *,
                    wd, nb, halo, cpad):
    j = pl.program_id(1)
    c, x = cur_ref.shape[1], cur_ref.shape[2]
    cout = o_ref.shape[1]

    prev_t = jnp.where(j == 0, 0.0, prev_ref[0, :, x - halo:])
    next_h = jnp.where(j == nb - 1, 0.0, next_ref[0, :, :halo])
    base = jnp.concatenate([prev_t, cur_ref[0], next_h], axis=1)
    if c < cpad:
        base = jnp.concatenate(
            [base, jnp.zeros((cpad - c, x + 2 * halo), jnp.float32)], axis=0)

    wcol = jax.lax.rem(jax.lax.broadcasted_iota(jnp.int32, (1, x), 1), wd)
    m_l, m_r = wcol == 0, wcol == wd - 1

    pieces = []
    for dy in (-1, 0, 1):
        for dx in (-1, 0, 1):
            s = halo + dy * wd + dx
            xs = base[:, s:s + x]
            if dx == -1:
                xs = jnp.where(m_l, 0.0, xs)
            elif dx == 1:
                xs = jnp.where(m_r, 0.0, xs)
            pieces.append(xs)
    pat = jnp.concatenate(pieces, axis=0)

    y = jax.lax.dot_general(w_ref[...], pat, (((0,), (0,)), ((), ())),
                            preferred_element_type=jnp.float32)

    nh2 = x // (2 * wd)
    ev = jnp.concatenate(
        [y[:, (2 * i) * wd:(2 * i + 1) * wd] for i in range(nh2)], axis=1)
    od = jnp.concatenate(
        [y[:, (2 * i + 1) * wd:(2 * i + 2) * wd] for i in range(nh2)], axis=1)
    o_ref[0] = jnp.maximum(jnp.maximum(ev, od) + b_ref[...], 0.0)


def _dec_plane_body(cur_ref, w_ref, b_ref, o_ref, *, wd):
    c, x = cur_ref.shape[1], cur_ref.shape[2]
    base = jnp.concatenate([cur_ref[0], jnp.zeros((c, 128), jnp.float32)],
                           axis=1)
    wcol = jax.lax.rem(jax.lax.broadcasted_iota(jnp.int32, (1, x), 1), wd)
    m_r = wcol == wd - 1

    pieces = []
    for ty in (0, 1):
        for tx in (0, 1):
            xs = base[:, ty * wd + tx: ty * wd + tx + x]
            if tx == 1:
                xs = jnp.where(m_r, 0.0, xs)
            pieces.append(xs)
    pat = jnp.concatenate(pieces, axis=0)

    r = jax.lax.dot_general(pat, w_ref[...], (((0,), (0,)), ((), ())),
                            preferred_element_type=jnp.float32)
    o_ref[0] = jnp.maximum(r + b_ref[...], 0.0)


def _dec_rows_body(cur_ref, next_ref, w_ref, b_ref, o_ref, *, wd, nb, act):
    blk = pl.program_id(1)
    bsz, cin = cur_ref.shape[1], cur_ref.shape[2]

    next_h = jnp.where(blk == nb - 1, 0.0, next_ref[0, :wd, :])
    zpad = jnp.zeros((8, cin), jnp.float32)
    xfull = jnp.concatenate([cur_ref[0], next_h, zpad], axis=0)

    wcol = jax.lax.rem(jax.lax.broadcasted_iota(jnp.int32, (bsz, 1), 0), wd)
    m_r = wcol == wd - 1

    pieces = []
    for ty in (0, 1):
        for tx in (0, 1):
            xs = xfull[ty * wd + tx: ty * wd + tx + bsz, :]
            if tx == 1:
                xs = jnp.where(m_r, 0.0, xs)
            pieces.append(xs)
    pat = jnp.concatenate(pieces, axis=1)

    r = jnp.dot(pat, w_ref[...], preferred_element_type=jnp.float32)
    r = r + b_ref[...]
    if act == "relu":
        r = jnp.maximum(r, 0.0)
    else:
        r = pl.reciprocal(1.0 + jnp.exp(-r), approx=True)
    o_ref[0] = r


def _enc_layer(x, w, b, wd, nb):
    n, c, hw = x.shape
    cout = w.shape[0]
    cpad = _rup(c, 8)
    halo = _rup(wd + 1, 128)
    wm, bb = _enc_w(w, b, cpad)
    xsz = hw // nb
    body = functools.partial(_enc_plane_body, wd=wd, nb=nb, halo=halo,
                             cpad=cpad)
    return pl.pallas_call(
        body,
        out_shape=jax.ShapeDtypeStruct((n, cout, hw // 2), jnp.float32),
        grid=(n, nb),
        in_specs=[
            pl.BlockSpec((1, c, xsz), lambda i, j: (i, 0, jnp.maximum(j - 1, 0))),
            pl.BlockSpec((1, c, xsz), lambda i, j: (i, 0, j)),
            pl.BlockSpec((1, c, xsz), lambda i, j: (i, 0, jnp.minimum(j + 1, nb - 1))),
            pl.BlockSpec((9 * cpad, cout), lambda i, j: (0, 0)),
            pl.BlockSpec((cout, 1), lambda i, j: (0, 0)),
        ],
        out_specs=pl.BlockSpec((1, cout, xsz // 2), lambda i, j: (i, 0, j)),
        compiler_params=pltpu.CompilerParams(
            dimension_semantics=("parallel", "arbitrary")),
    )(x, x, x, wm, bb)


def _dec_plane_layer(x, w, b, wd):
    n, c, hw = x.shape
    cout = w.shape[1]
    wm, bb = _dec_w(w, b)
    body = functools.partial(_dec_plane_body, wd=wd)
    return pl.pallas_call(
        body,
        out_shape=jax.ShapeDtypeStruct((n, hw, 4 * cout), jnp.float32),
        grid=(n,),
        in_specs=[
            pl.BlockSpec((1, c, hw), lambda i: (i, 0, 0)),
            pl.BlockSpec((4 * c, 4 * cout), lambda i: (0, 0)),
            pl.BlockSpec((1, 4 * cout), lambda i: (0, 0)),
        ],
        out_specs=pl.BlockSpec((1, hw, 4 * cout), lambda i: (i, 0, 0)),
        compiler_params=pltpu.CompilerParams(
            dimension_semantics=("parallel",)),
    )(x, wm, bb)


def _dec_rows_layer(x, w, b, wd, act, nb):
    n, hw, cin = x.shape
    cout = w.shape[1]
    wm, bb = _dec_w(w, b)
    bsz = hw // nb
    body = functools.partial(_dec_rows_body, wd=wd, nb=nb, act=act)
    return pl.pallas_call(
        body,
        out_shape=jax.ShapeDtypeStruct((n, hw, 4 * cout), jnp.float32),
        grid=(n, nb),
        in_specs=[
            pl.BlockSpec((1, bsz, cin), lambda i, j: (i, j, 0)),
            pl.BlockSpec((1, bsz, cin), lambda i, j: (i, jnp.minimum(j + 1, nb - 1), 0)),
            pl.BlockSpec((4 * cin, 4 * cout), lambda i, j: (0, 0)),
            pl.BlockSpec((1, 4 * cout), lambda i, j: (0, 0)),
        ],
        out_specs=pl.BlockSpec((1, bsz, 4 * cout), lambda i, j: (i, j, 0)),
        compiler_params=pltpu.CompilerParams(
            dimension_semantics=("parallel", "arbitrary")),
    )(x, x, wm, bb)


def _interleave(r, hgrid, wgrid):
    n = r.shape[0]
    co = r.shape[2] // 4
    y = r.reshape(n, hgrid, wgrid, 2, 2, co)
    y = jnp.transpose(y, (0, 1, 3, 2, 4, 5))
    return y.reshape(n, 4 * hgrid * wgrid, co)


def kernel(x, enc1_w, enc1_b, enc2_w, enc2_b, enc3_w, enc3_b,
           dec1_w, dec1_b, dec2_w, dec2_b, dec3_w, dec3_b):
    n, cin, h, w = x.shape
    xp = x.reshape(n, cin, h * w)

    def wmax(t):
        return jnp.max(t.reshape(n, t.shape[1], t.shape[2] // 2, 2), axis=3)

    e1 = wmax(_enc_layer(xp, enc1_w, enc1_b, wd=w, nb=4))
    e2 = wmax(_enc_layer(e1, enc2_w, enc2_b, wd=w // 2, nb=2))
    e3 = wmax(_enc_layer(e2, enc3_w, enc3_b, wd=w // 4, nb=1))

    d1 = _dec_plane_layer(e3, dec1_w, dec1_b, wd=w // 8)
    d1 = _interleave(d1, h // 8, w // 8)
    d2 = _dec_rows_layer(d1, dec2_w, dec2_b, wd=w // 4, act="relu", nb=1)
    d2 = _interleave(d2, h // 4, w // 4)
    d3 = _dec_rows_layer(d2, dec3_w, dec3_b, wd=w // 2, act="sigmoid", nb=4)

    h2, w2, co = h // 2, w // 2, dec3_w.shape[1]
    y = d3.reshape(n, h2, w2, 2, 2, co)
    return jnp.transpose(y, (0, 5, 1, 3, 2, 4)).reshape(n, co, h, w)


# speedup vs baseline: 7.2548x; 1.0964x over previous
import functools

import jax
import jax.numpy as jnp
from jax.experimental import pallas as pl
from jax.experimental.pallas import tpu as pltpu


def _rup(x, m):
    return (x + m - 1) // m * m


def _enc_w(w, b, cpad):
    cout, cin = w.shape[0], w.shape[1]
    wm = jnp.transpose(w, (2, 3, 1, 0)).astype(jnp.float32)
    wm = jnp.pad(wm, ((0, 0), (0, 0), (0, cpad - cin), (0, 0)))
    return wm.reshape(9 * cpad, cout).astype(jnp.bfloat16), b.astype(jnp.float32).reshape(cout, 1)


def _dec_w(w, b):
    cin, cout = w.shape[0], w.shape[1]
    z = jnp.zeros((cin, cout), w.dtype)
    rows = []
    for ty in range(2):
        for tx in range(2):
            blocks = []
            for py in range(2):
                for px in range(2):
                    kh, kw = py - 2 * ty + 1, px - 2 * tx + 1
                    ok = 0 <= kh <= 2 and 0 <= kw <= 2
                    blocks.append(w[:, :, kh, kw] if ok else z)
            rows.append(jnp.concatenate(blocks, axis=1))
    wm = jnp.concatenate(rows, axis=0).astype(jnp.bfloat16)
    bb = jnp.tile(b.astype(jnp.float32), 4).reshape(1, 4 * cout)
    return wm, bb


def _enc_plane_body(prev_ref, cur_ref, next_ref, w_ref, b_ref, o_ref, *,
                    wd, nb, halo, cpad):
    j = pl.program_id(1)
    c, x = cur_ref.shape[1], cur_ref.shape[2]
    cout = o_ref.shape[1]

    prev_t = jnp.where(j == 0, 0.0, prev_ref[0, :, x - halo:])
    next_h = jnp.where(j == nb - 1, 0.0, next_ref[0, :, :halo])
    base = jnp.concatenate([prev_t, cur_ref[0], next_h], axis=1)
    base = base.astype(jnp.bfloat16)
    if c < cpad:
        base = jnp.concatenate(
            [base, jnp.zeros((cpad - c, x + 2 * halo), jnp.bfloat16)], axis=0)

    wcol = jax.lax.rem(jax.lax.broadcasted_iota(jnp.int32, (1, x), 1), wd)
    m_l, m_r = wcol == 0, wcol == wd - 1

    pieces = []
    for dy in (-1, 0, 1):
        for dx in (-1, 0, 1):
            s = halo + dy * wd + dx
            xs = base[:, s:s + x]
            if dx == -1:
                xs = jnp.where(m_l, 0.0, xs)
            elif dx == 1:
                xs = jnp.where(m_r, 0.0, xs)
            pieces.append(xs)
    pat = jnp.concatenate(pieces, axis=0)

    y = jax.lax.dot_general(w_ref[...], pat, (((0,), (0,)), ((), ())),
                            preferred_element_type=jnp.float32)

    nh2 = x // (2 * wd)
    ev = jnp.concatenate(
        [y[:, (2 * i) * wd:(2 * i + 1) * wd] for i in range(nh2)], axis=1)
    od = jnp.concatenate(
        [y[:, (2 * i + 1) * wd:(2 * i + 2) * wd] for i in range(nh2)], axis=1)
    o_ref[0] = jnp.maximum(jnp.maximum(ev, od) + b_ref[...], 0.0).astype(o_ref.dtype)


def _dec_plane_body(cur_ref, w_ref, b_ref, o_ref, *, wd):
    c, x = cur_ref.shape[1], cur_ref.shape[2]
    base = jnp.concatenate(
        [cur_ref[0], jnp.zeros((c, 128), cur_ref.dtype)], axis=1)
    wcol = jax.lax.rem(jax.lax.broadcasted_iota(jnp.int32, (1, x), 1), wd)
    m_r = wcol == wd - 1

    pieces = []
    for ty in (0, 1):
        for tx in (0, 1):
            xs = base[:, ty * wd + tx: ty * wd + tx + x]
            if tx == 1:
                xs = jnp.where(m_r, 0.0, xs)
            pieces.append(xs)
    pat = jnp.concatenate(pieces, axis=0)

    r = jax.lax.dot_general(pat, w_ref[...], (((0,), (0,)), ((), ())),
                            preferred_element_type=jnp.float32)
    o_ref[0] = jnp.maximum(r + b_ref[...], 0.0).astype(o_ref.dtype)


def _dec_rows_body(cur_ref, next_ref, w_ref, b_ref, o_ref, *, wd, nb, act):
    blk = pl.program_id(1)
    bsz, cin = cur_ref.shape[1], cur_ref.shape[2]

    next_h = jnp.where(blk == nb - 1, 0.0, next_ref[0, :wd, :])
    zpad = jnp.zeros((8, cin), cur_ref.dtype)
    xfull = jnp.concatenate([cur_ref[0], next_h, zpad], axis=0)

    wcol = jax.lax.rem(jax.lax.broadcasted_iota(jnp.int32, (bsz, 1), 0), wd)
    m_r = wcol == wd - 1

    pieces = []
    for ty in (0, 1):
        for tx in (0, 1):
            xs = xfull[ty * wd + tx: ty * wd + tx + bsz, :]
            if tx == 1:
                xs = jnp.where(m_r, 0.0, xs)
            pieces.append(xs)
    pat = jnp.concatenate(pieces, axis=1)

    r = jnp.dot(pat, w_ref[...], preferred_element_type=jnp.float32)
    r = r + b_ref[...]
    if act == "relu":
        r = jnp.maximum(r, 0.0)
    else:
        r = pl.reciprocal(1.0 + jnp.exp(-r), approx=True)
    o_ref[0] = r.astype(o_ref.dtype)


def _enc_layer(x, w, b, wd, nb):
    n, c, hw = x.shape
    cout = w.shape[0]
    cpad = _rup(c, 8)
    halo = _rup(wd + 1, 128)
    wm, bb = _enc_w(w, b, cpad)
    xsz = hw // nb
    body = functools.partial(_enc_plane_body, wd=wd, nb=nb, halo=halo,
                             cpad=cpad)
    return pl.pallas_call(
        body,
        out_shape=jax.ShapeDtypeStruct((n, cout, hw // 2), jnp.bfloat16),
        grid=(n, nb),
        in_specs=[
            pl.BlockSpec((1, c, xsz), lambda i, j: (i, 0, jnp.maximum(j - 1, 0))),
            pl.BlockSpec((1, c, xsz), lambda i, j: (i, 0, j)),
            pl.BlockSpec((1, c, xsz), lambda i, j: (i, 0, jnp.minimum(j + 1, nb - 1))),
            pl.BlockSpec((9 * cpad, cout), lambda i, j: (0, 0)),
            pl.BlockSpec((cout, 1), lambda i, j: (0, 0)),
        ],
        out_specs=pl.BlockSpec((1, cout, xsz // 2), lambda i, j: (i, 0, j)),
        compiler_params=pltpu.CompilerParams(
            dimension_semantics=("parallel", "arbitrary")),
    )(x, x, x, wm, bb)


def _dec_plane_layer(x, w, b, wd):
    n, c, hw = x.shape
    cout = w.shape[1]
    wm, bb = _dec_w(w, b)
    body = functools.partial(_dec_plane_body, wd=wd)
    return pl.pallas_call(
        body,
        out_shape=jax.ShapeDtypeStruct((n, hw, 4 * cout), jnp.bfloat16),
        grid=(n,),
        in_specs=[
            pl.BlockSpec((1, c, hw), lambda i: (i, 0, 0)),
            pl.BlockSpec((4 * c, 4 * cout), lambda i: (0, 0)),
            pl.BlockSpec((1, 4 * cout), lambda i: (0, 0)),
        ],
        out_specs=pl.BlockSpec((1, hw, 4 * cout), lambda i: (i, 0, 0)),
        compiler_params=pltpu.CompilerParams(
            dimension_semantics=("parallel",)),
    )(x, wm, bb)


def _dec_rows_layer(x, w, b, wd, act, nb, out_dtype=jnp.bfloat16):
    n, hw, cin = x.shape
    cout = w.shape[1]
    wm, bb = _dec_w(w, b)
    bsz = hw // nb
    body = functools.partial(_dec_rows_body, wd=wd, nb=nb, act=act)
    return pl.pallas_call(
        body,
        out_shape=jax.ShapeDtypeStruct((n, hw, 4 * cout), out_dtype),
        grid=(n, nb),
        in_specs=[
            pl.BlockSpec((1, bsz, cin), lambda i, j: (i, j, 0)),
            pl.BlockSpec((1, bsz, cin), lambda i, j: (i, jnp.minimum(j + 1, nb - 1), 0)),
            pl.BlockSpec((4 * cin, 4 * cout), lambda i, j: (0, 0)),
            pl.BlockSpec((1, 4 * cout), lambda i, j: (0, 0)),
        ],
        out_specs=pl.BlockSpec((1, bsz, 4 * cout), lambda i, j: (i, j, 0)),
        compiler_params=pltpu.CompilerParams(
            dimension_semantics=("parallel", "arbitrary")),
    )(x, x, wm, bb)


def _interleave(r, hgrid, wgrid):
    n = r.shape[0]
    co = r.shape[2] // 4
    y = r.reshape(n, hgrid, wgrid, 2, 2, co)
    y = jnp.transpose(y, (0, 1, 3, 2, 4, 5))
    return y.reshape(n, 4 * hgrid * wgrid, co)


def kernel(x, enc1_w, enc1_b, enc2_w, enc2_b, enc3_w, enc3_b,
           dec1_w, dec1_b, dec2_w, dec2_b, dec3_w, dec3_b):
    n, cin, h, w = x.shape
    xp = x.reshape(n, cin, h * w)

    def wmax(t):
        return jnp.max(t.reshape(n, t.shape[1], t.shape[2] // 2, 2), axis=3)

    e1 = wmax(_enc_layer(xp, enc1_w, enc1_b, wd=w, nb=4))
    e2 = wmax(_enc_layer(e1, enc2_w, enc2_b, wd=w // 2, nb=2))
    e3 = wmax(_enc_layer(e2, enc3_w, enc3_b, wd=w // 4, nb=1))

    d1 = _dec_plane_layer(e3, dec1_w, dec1_b, wd=w // 8)
    d1 = _interleave(d1, h // 8, w // 8)
    d2 = _dec_rows_layer(d1, dec2_w, dec2_b, wd=w // 4, act="relu", nb=1)
    d2 = _interleave(d2, h // 4, w // 4)
    d3 = _dec_rows_layer(d2, dec3_w, dec3_b, wd=w // 2, act="sigmoid", nb=4,
                         out_dtype=jnp.float32)

    h2, w2, co = h // 2, w // 2, dec3_w.shape[1]
    y = d3.reshape(n, h2, w2, 2, 2, co)
    return jnp.transpose(y, (0, 5, 1, 3, 2, 4)).reshape(n, co, h, w)
